# baseline (speedup 1.0000x reference)
import functools

import jax
import jax.numpy as jnp
from jax import lax
from jax.experimental import pallas as pl
from jax.experimental.pallas import tpu as pltpu
from jax.experimental.pallas import tpu_sc as plsc

N = 10000
D = 128
E = 320000

NC = 2
NS = 16
NW = NC * NS
CHUNK = 32
N_CHUNKS = E // CHUNK
PADC = 10240
WSPAN = PADC // NW
MAXC = WSPAN
NBUF = 8
HALF = WSPAN // 8
NPHASE = WSPAN // HALF
ZR = 8

_mesh = plsc.VectorSubcoreMesh(core_axis_name="c", subcore_axis_name="s")

_LANE_ZERO = functools.partial(jnp.zeros, (16,), jnp.float32)
_LANE_ONE = functools.partial(jnp.ones, (16,), jnp.float32)


def _worker_span(wid):
    r0 = wid * WSPAN
    nch = jnp.minimum(WSPAN, N_CHUNKS - r0)
    return r0, nch


@functools.partial(
    pl.kernel,
    out_type=jax.ShapeDtypeStruct((NC * N,), jnp.float32),
    mesh=_mesh,
    scratch_types=[
        pltpu.VMEM_SHARED((N,), jnp.float32),
        pltpu.VMEM((80,), jnp.float32),
        pltpu.VMEM((CHUNK,), jnp.float32),
        pltpu.VMEM((MAXC, CHUNK), jnp.int32),
        pltpu.SemaphoreType.DMA,
    ],
)
def _deg_kernel(dst2_hbm, deg_hbm, acc_s, zbuf_v, ones_v, didx_v, sem):
    c = lax.axis_index("c")
    s = lax.axis_index("s")
    wid = c * NS + s
    r0, nch = _worker_span(wid)

    for j in range(80 // 16):
        zbuf_v[pl.ds(j * 16, 16)] = _LANE_ZERO()
    for j in range(CHUNK // 16):
        ones_v[pl.ds(j * 16, 16)] = _LANE_ONE()

    pltpu.sync_copy(dst2_hbm.at[pl.ds(r0, MAXC)], didx_v)

    def zbody(k, carry):
        rc = s + NS * k

        @pl.when(rc < N // 80)
        def _():
            pltpu.sync_copy(zbuf_v, acc_s.at[pl.ds(rc * 80, 80)])

        return carry

    lax.fori_loop(0, -(-(N // 80) // NS), zbody, 0)
    plsc.subcore_barrier()

    Q = 8

    def ebody(i, carry):
        @pl.when(i < nch)
        def _():
            pltpu.async_copy(ones_v, acc_s.at[didx_v.at[i]], sem, add=True)

        @pl.when((i >= Q) & (i - Q < nch))
        def _():
            pltpu.make_async_copy(
                ones_v, acc_s.at[didx_v.at[jnp.maximum(i - Q, 0)]], sem).wait()

        return carry

    lax.fori_loop(0, MAXC + Q, ebody, 0)
    plsc.subcore_barrier()

    def wbody(k, carry):
        rc = s + NS * k

        @pl.when(rc < N // 80)
        def _():
            pltpu.sync_copy(acc_s.at[pl.ds(rc * 80, 80)], zbuf_v)
            pltpu.sync_copy(zbuf_v, deg_hbm.at[pl.ds(c * N + rc * 80, 80)])

        return carry

    lax.fori_loop(0, -(-(N // 80) // NS), wbody, 0)


@functools.partial(
    pl.kernel,
    out_type=jax.ShapeDtypeStruct((NC, N, D), jnp.float32),
    mesh=_mesh,
    scratch_types=[
        pltpu.VMEM_SHARED((N, D), jnp.float32),
        pltpu.VMEM((ZR, D), jnp.float32),
        pltpu.VMEM((HALF, CHUNK), jnp.int32),
        pltpu.VMEM((HALF, CHUNK), jnp.int32),
        [pltpu.VMEM((CHUNK, D), jnp.float32)] * NBUF,
        [pltpu.SemaphoreType.DMA] * NBUF,
        [pltpu.SemaphoreType.DMA] * NBUF,
    ],
)
def _mp_kernel(src2_hbm, dst2_hbm, g_hbm, outs_hbm, acc_s, zrow_v, sidx_v,
               didx_v, rows, gsem, ssem):
    c = lax.axis_index("c")
    s = lax.axis_index("s")
    wid = c * NS + s
    oc = outs_hbm.at[c]
    r0, nch = _worker_span(wid)

    for r in range(ZR):
        for j in range(D // 16):
            zrow_v[r, pl.ds(j * 16, 16)] = _LANE_ZERO()

    nrc = N // ZR

    def zbody(k, carry):
        rc = s + NS * k

        @pl.when(rc < nrc)
        def _():
            pltpu.sync_copy(zrow_v, acc_s.at[pl.ds(rc * ZR, ZR)])

        return carry

    lax.fori_loop(0, -(-nrc // NS), zbody, 0)
    plsc.subcore_barrier()

    def gather(j, slot):
        return pltpu.make_async_copy(g_hbm.at[sidx_v.at[j]], rows[slot],
                                     gsem[slot])

    def scatter(j, slot):
        return pltpu.make_async_copy(rows[slot], acc_s.at[didx_v.at[j]],
                                     ssem[slot])

    for phase in range(NPHASE):
        base = r0 + phase * HALF
        nph = jnp.clip(nch - phase * HALF, 0, HALF)
        pltpu.sync_copy(src2_hbm.at[pl.ds(base, HALF)], sidx_v)
        pltpu.sync_copy(dst2_hbm.at[pl.ds(base, HALF)], didx_v)

        for j in range(NBUF - 1):
            @pl.when(j < nph)
            def _():
                gather(j, j).start()

        def ebody(step, carry):
            for u in range(NBUF):
                i = step * NBUF + u
                slot = u
                gslot = (u + NBUF - 1) % NBUF

                @pl.when((i >= 1) & (i < nph))
                def _():
                    scatter(jnp.maximum(i - 1, 0), gslot).wait()

                @pl.when(i + NBUF - 1 < nph)
                def _():
                    gather(i + NBUF - 1, gslot).start()

                @pl.when(i < nph)
                def _():
                    gather(i, slot).wait()
                    pltpu.async_copy(rows[slot], acc_s.at[didx_v.at[i]],
                                     ssem[slot], add=True)

            return carry

        lax.fori_loop(0, HALF // NBUF, ebody, 0)

        for u in range(NBUF):
            @pl.when((nph >= 1) & ((nph - 1) % NBUF == u))
            def _():
                scatter(nph - 1, u).wait()

    plsc.subcore_barrier()

    def wbody(k, carry):
        j = s + NS * k

        @pl.when(j < N // CHUNK)
        def _():
            pltpu.sync_copy(acc_s.at[pl.ds(j * CHUNK, CHUNK)], rows[0])
            pltpu.sync_copy(rows[0], oc.at[pl.ds(j * CHUNK, CHUNK)])

        return carry

    lax.fori_loop(0, -(-(N // CHUNK) // NS), wbody, 0)

    @pl.when(s == 0)
    def _():
        tail = (N // CHUNK) * CHUNK
        pltpu.sync_copy(acc_s.at[pl.ds(tail, N - tail)],
                        rows[0].at[pl.ds(0, N - tail)])
        pltpu.sync_copy(rows[0].at[pl.ds(0, N - tail)],
                        oc.at[pl.ds(tail, N - tail)])


BM = 1000


def _tc1_body(x_ref, w_ref, dega_ref, degb_ref, g_ref, dinv_ref):
    deg = dega_ref[...] + degb_ref[...] + 1.0
    dinv = lax.rsqrt(deg)
    h = jnp.dot(x_ref[...], w_ref[...], preferred_element_type=jnp.float32)
    g_ref[...] = h * dinv
    dinv_ref[...] = dinv


def _tc_mid_body(acc_ref, g_ref, dinv_ref, b_ref, w_ref, gout_ref):
    dinv = dinv_ref[...]
    z = (acc_ref[0] + acc_ref[1] + g_ref[...]) * dinv + b_ref[...]
    a = jnp.maximum(z, 0.0)
    gout_ref[...] = (
        jnp.dot(a, w_ref[...], preferred_element_type=jnp.float32) * dinv)


def _tc_fin_body(acc_ref, g_ref, dinv_ref, b_ref, out_ref):
    out_ref[...] = ((acc_ref[0] + acc_ref[1] + g_ref[...]) * dinv_ref[...]
                    + b_ref[...])


_row_spec = pl.BlockSpec((BM, D), lambda i: (i, 0))
_pair_spec = pl.BlockSpec((2, BM, D), lambda i: (0, i, 0))
_col_spec = pl.BlockSpec((BM, 1), lambda i: (i, 0))
_w_spec = pl.BlockSpec((D, D), lambda i: (0, 0))
_b_spec = pl.BlockSpec((1, D), lambda i: (0, 0))

_tc1 = pl.pallas_call(
    _tc1_body,
    grid=(N // BM,),
    in_specs=[_row_spec, _w_spec, _col_spec, _col_spec],
    out_specs=[_row_spec, _col_spec],
    out_shape=[jax.ShapeDtypeStruct((N, D), jnp.float32),
               jax.ShapeDtypeStruct((N, 1), jnp.float32)],
)

_tc_mid = pl.pallas_call(
    _tc_mid_body,
    grid=(N // BM,),
    in_specs=[_pair_spec, _row_spec, _col_spec, _b_spec, _w_spec],
    out_specs=_row_spec,
    out_shape=jax.ShapeDtypeStruct((N, D), jnp.float32),
)

_tc_fin = pl.pallas_call(
    _tc_fin_body,
    grid=(N // BM,),
    in_specs=[_pair_spec, _row_spec, _col_spec, _b_spec],
    out_specs=_row_spec,
    out_shape=jax.ShapeDtypeStruct((N, D), jnp.float32),
)


def kernel(x, edge_index, W1, b1, W2, b2, W3, b3):
    src = edge_index[0].astype(jnp.int32)
    dst = edge_index[1].astype(jnp.int32)
    src2 = jnp.pad(src.reshape(N_CHUNKS, CHUNK), ((0, PADC - N_CHUNKS), (0, 0)))
    dst2 = jnp.pad(dst.reshape(N_CHUNKS, CHUNK), ((0, PADC - N_CHUNKS), (0, 0)))

    degp = _deg_kernel(dst2)
    dega = degp[:N].reshape(N, 1)
    degb = degp[N:].reshape(N, 1)

    g1, dinv = _tc1(x, W1, dega, degb)
    acc1 = _mp_kernel(src2, dst2, g1)
    g2 = _tc_mid(acc1, g1, dinv, b1.reshape(1, D), W2)
    acc2 = _mp_kernel(src2, dst2, g2)
    g3 = _tc_mid(acc2, g2, dinv, b2.reshape(1, D), W3)
    acc3 = _mp_kernel(src2, dst2, g3)
    return _tc_fin(acc3, g3, dinv, b3.reshape(1, D))


# speedup vs baseline: 1.1847x; 1.1847x over previous
import functools

import jax
import jax.numpy as jnp
from jax import lax
from jax.experimental import pallas as pl
from jax.experimental.pallas import tpu as pltpu
from jax.experimental.pallas import tpu_sc as plsc

N = 10000
D = 128
E = 320000

NC = 2
NS = 16
NW = NC * NS
CHUNK = 64
N_CHUNKS = E // CHUNK
WSPAN = 160
NBUF = 4
HALF = WSPAN // 4
NPHASE = WSPAN // HALF
ZR = 8

_mesh = plsc.VectorSubcoreMesh(core_axis_name="c", subcore_axis_name="s")

_LANE_ZERO = functools.partial(jnp.zeros, (16,), jnp.float32)
_LANE_ONE = functools.partial(jnp.ones, (16,), jnp.float32)


def _worker_span(wid):
    r0 = wid * WSPAN
    nch = jnp.minimum(WSPAN, N_CHUNKS - r0)
    return r0, nch


@functools.partial(
    pl.kernel,
    out_type=jax.ShapeDtypeStruct((NC * N,), jnp.float32),
    mesh=_mesh,
    scratch_types=[
        pltpu.VMEM_SHARED((N,), jnp.float32),
        pltpu.VMEM((80,), jnp.float32),
        pltpu.VMEM((CHUNK,), jnp.float32),
        pltpu.VMEM((HALF, CHUNK), jnp.int32),
        pltpu.SemaphoreType.DMA,
    ],
)
def _deg_kernel(dst2_hbm, deg_hbm, acc_s, zbuf_v, ones_v, didx_v, sem):
    c = lax.axis_index("c")
    s = lax.axis_index("s")
    wid = c * NS + s
    r0, nch = _worker_span(wid)

    for j in range(80 // 16):
        zbuf_v[pl.ds(j * 16, 16)] = _LANE_ZERO()
    for j in range(CHUNK // 16):
        ones_v[pl.ds(j * 16, 16)] = _LANE_ONE()

    def zbody(k, carry):
        rc = s + NS * k

        @pl.when(rc < N // 80)
        def _():
            pltpu.sync_copy(zbuf_v, acc_s.at[pl.ds(rc * 80, 80)])

        return carry

    lax.fori_loop(0, -(-(N // 80) // NS), zbody, 0)
    plsc.subcore_barrier()

    Q = 8

    for phase in range(NPHASE):
        base = r0 + phase * HALF
        nph = jnp.clip(nch - phase * HALF, 0, HALF)

        @pl.when(nph > 0)
        def _():
            pltpu.sync_copy(dst2_hbm.at[pl.ds(base, HALF)], didx_v)

        def ebody(i, carry):
            @pl.when(i < nph)
            def _():
                pltpu.async_copy(ones_v, acc_s.at[didx_v.at[i]], sem,
                                 add=True)

            @pl.when((i >= Q) & (i - Q < nph))
            def _():
                pltpu.make_async_copy(
                    ones_v, acc_s.at[didx_v.at[jnp.maximum(i - Q, 0)]],
                    sem).wait()

            return carry

        lax.fori_loop(0, HALF + Q, ebody, 0)

    plsc.subcore_barrier()

    def wbody(k, carry):
        rc = s + NS * k

        @pl.when(rc < N // 80)
        def _():
            pltpu.sync_copy(acc_s.at[pl.ds(rc * 80, 80)], zbuf_v)
            pltpu.sync_copy(zbuf_v, deg_hbm.at[pl.ds(c * N + rc * 80, 80)])

        return carry

    lax.fori_loop(0, -(-(N // 80) // NS), wbody, 0)


@functools.partial(
    pl.kernel,
    out_type=jax.ShapeDtypeStruct((NC, N, D), jnp.float32),
    mesh=_mesh,
    scratch_types=[
        pltpu.VMEM_SHARED((N, D), jnp.float32),
        pltpu.VMEM((HALF, CHUNK), jnp.int32),
        pltpu.VMEM((HALF, CHUNK), jnp.int32),
        [pltpu.VMEM((CHUNK, D), jnp.float32)] * NBUF,
        [pltpu.SemaphoreType.DMA] * NBUF,
        [pltpu.SemaphoreType.DMA] * NBUF,
    ],
)
def _mp_kernel(src2_hbm, dst2_hbm, g_hbm, outs_hbm, acc_s, sidx_v,
               didx_v, rows, gsem, ssem):
    c = lax.axis_index("c")
    s = lax.axis_index("s")
    wid = c * NS + s
    oc = outs_hbm.at[c]
    r0, nch = _worker_span(wid)

    for r in range(CHUNK):
        for j in range(D // 16):
            rows[0][r, pl.ds(j * 16, 16)] = _LANE_ZERO()

    nzc = N // CHUNK

    def zbody(k, carry):
        rc = s + NS * k

        @pl.when(rc < nzc)
        def _():
            pltpu.sync_copy(rows[0], acc_s.at[pl.ds(rc * CHUNK, CHUNK)])

        return carry

    lax.fori_loop(0, -(-nzc // NS), zbody, 0)

    @pl.when(s == 0)
    def _():
        ztail = nzc * CHUNK
        pltpu.sync_copy(rows[0].at[pl.ds(0, N - ztail)],
                        acc_s.at[pl.ds(ztail, N - ztail)])

    plsc.subcore_barrier()

    def gather(j, slot):
        return pltpu.make_async_copy(g_hbm.at[sidx_v.at[j]], rows[slot],
                                     gsem[slot])

    def scatter(j, slot):
        return pltpu.make_async_copy(rows[slot], acc_s.at[didx_v.at[j]],
                                     ssem[slot])

    for phase in range(NPHASE):
        base = r0 + phase * HALF
        nph = jnp.clip(nch - phase * HALF, 0, HALF)

        @pl.when(nph > 0)
        def _():
            pltpu.sync_copy(src2_hbm.at[pl.ds(base, HALF)], sidx_v)
            pltpu.sync_copy(dst2_hbm.at[pl.ds(base, HALF)], didx_v)

        for j in range(NBUF - 1):
            @pl.when(j < nph)
            def _():
                gather(j, j).start()

        def ebody(step, carry):
            for u in range(NBUF):
                i = step * NBUF + u
                slot = u
                gslot = (u + NBUF - 1) % NBUF

                @pl.when((i >= 1) & (i < nph))
                def _():
                    scatter(jnp.maximum(i - 1, 0), gslot).wait()

                @pl.when(i + NBUF - 1 < nph)
                def _():
                    gather(i + NBUF - 1, gslot).start()

                @pl.when(i < nph)
                def _():
                    gather(i, slot).wait()
                    pltpu.async_copy(rows[slot], acc_s.at[didx_v.at[i]],
                                     ssem[slot], add=True)

            return carry

        lax.fori_loop(0, HALF // NBUF, ebody, 0)

        for u in range(NBUF):
            @pl.when((nph >= 1) & ((nph - 1) % NBUF == u))
            def _():
                scatter(nph - 1, u).wait()

    plsc.subcore_barrier()

    nzc2 = N // CHUNK

    for k in range(-(-nzc2 // NS)):
        j = s + NS * k
        slot = k % 2

        if k >= 2:
            jp = s + NS * (k - 2)

            @pl.when(jp < nzc2)
            def _():
                pltpu.make_async_copy(
                    rows[slot], oc.at[pl.ds(jp * CHUNK, CHUNK)],
                    gsem[slot]).wait()

        @pl.when(j < nzc2)
        def _():
            pltpu.sync_copy(acc_s.at[pl.ds(j * CHUNK, CHUNK)], rows[slot])
            pltpu.async_copy(rows[slot], oc.at[pl.ds(j * CHUNK, CHUNK)],
                             gsem[slot])

    for k in range(-(-nzc2 // NS) - 2, -(-nzc2 // NS)):
        j = s + NS * k
        slot = k % 2

        @pl.when(j < nzc2)
        def _():
            pltpu.make_async_copy(
                rows[slot], oc.at[pl.ds(j * CHUNK, CHUNK)], gsem[slot]).wait()

    @pl.when(s == 0)
    def _():
        tail = nzc2 * CHUNK
        pltpu.sync_copy(acc_s.at[pl.ds(tail, N - tail)],
                        rows[2].at[pl.ds(0, N - tail)])
        pltpu.sync_copy(rows[2].at[pl.ds(0, N - tail)],
                        oc.at[pl.ds(tail, N - tail)])


BM = 1000


def _tc1_body(x_ref, w_ref, dega_ref, degb_ref, g_ref, dinv_ref):
    deg = dega_ref[...] + degb_ref[...] + 1.0
    dinv = lax.rsqrt(deg)
    h = jnp.dot(x_ref[...], w_ref[...], preferred_element_type=jnp.float32)
    g_ref[...] = h * dinv
    dinv_ref[...] = dinv


def _tc_mid_body(acc_ref, g_ref, dinv_ref, b_ref, w_ref, gout_ref):
    dinv = dinv_ref[...]
    z = (acc_ref[0] + acc_ref[1] + g_ref[...]) * dinv + b_ref[...]
    a = jnp.maximum(z, 0.0)
    gout_ref[...] = (
        jnp.dot(a, w_ref[...], preferred_element_type=jnp.float32) * dinv)


def _tc_fin_body(acc_ref, g_ref, dinv_ref, b_ref, out_ref):
    out_ref[...] = ((acc_ref[0] + acc_ref[1] + g_ref[...]) * dinv_ref[...]
                    + b_ref[...])


_row_spec = pl.BlockSpec((BM, D), lambda i: (i, 0))
_pair_spec = pl.BlockSpec((2, BM, D), lambda i: (0, i, 0))
_col_spec = pl.BlockSpec((BM, 1), lambda i: (i, 0))
_w_spec = pl.BlockSpec((D, D), lambda i: (0, 0))
_b_spec = pl.BlockSpec((1, D), lambda i: (0, 0))

_tc1 = pl.pallas_call(
    _tc1_body,
    grid=(N // BM,),
    in_specs=[_row_spec, _w_spec, _col_spec, _col_spec],
    out_specs=[_row_spec, _col_spec],
    out_shape=[jax.ShapeDtypeStruct((N, D), jnp.float32),
               jax.ShapeDtypeStruct((N, 1), jnp.float32)],
)

_tc_mid = pl.pallas_call(
    _tc_mid_body,
    grid=(N // BM,),
    in_specs=[_pair_spec, _row_spec, _col_spec, _b_spec, _w_spec],
    out_specs=_row_spec,
    out_shape=jax.ShapeDtypeStruct((N, D), jnp.float32),
)

_tc_fin = pl.pallas_call(
    _tc_fin_body,
    grid=(N // BM,),
    in_specs=[_pair_spec, _row_spec, _col_spec, _b_spec],
    out_specs=_row_spec,
    out_shape=jax.ShapeDtypeStruct((N, D), jnp.float32),
)


def kernel(x, edge_index, W1, b1, W2, b2, W3, b3):
    src = edge_index[0].astype(jnp.int32)
    dst = edge_index[1].astype(jnp.int32)
    src2 = src.reshape(N_CHUNKS, CHUNK)
    dst2 = dst.reshape(N_CHUNKS, CHUNK)

    degp = _deg_kernel(dst2)
    dega = degp[:N].reshape(N, 1)
    degb = degp[N:].reshape(N, 1)

    g1, dinv = _tc1(x, W1, dega, degb)
    acc1 = _mp_kernel(src2, dst2, g1)
    g2 = _tc_mid(acc1, g1, dinv, b1.reshape(1, D), W2)
    acc2 = _mp_kernel(src2, dst2, g2)
    g3 = _tc_mid(acc2, g2, dinv, b2.reshape(1, D), W3)
    acc3 = _mp_kernel(src2, dst2, g3)
    return _tc_fin(acc3, g3, dinv, b3.reshape(1, D))
